# baseline (speedup 1.0000x reference)
import functools

import jax
import jax.numpy as jnp
from jax import lax
from jax.experimental import pallas as pl
from jax.experimental.pallas import tpu as pltpu
from jax.experimental.pallas import tpu_sc as plsc

N_NODES = 10000
D = 128

NC = 2
NS = 16
NW = NC * NS

CH = 72
SUB = 24
K_SLOW = 6
K_FAST = 6
SLOW_CORE = 1
WCH = 64
N_CH_TOT = NS * (K_SLOW + K_FAST) * SUB
E_PAD = N_CH_TOT * CH
R_ACC = 10240
R_T = R_ACC // NS
DUMMY = N_NODES


def _agg_body(x_hbm, srcc_hbm, dstc_hbm, out_hbm,
              is_a, is_b, id_a, id_b, gbuf_a, gbuf_b, gbuf_c, acc_sh,
              sem_a, sem_b, sem_c, sem_i):
    c = lax.axis_index("c")
    s = lax.axis_index("s")
    sup0 = jnp.where(c == SLOW_CORE, s * K_SLOW,
                     NS * K_SLOW + s * K_FAST)

    def zero_body(i, _):
        for cc in range(D // 16):
            gbuf_a[i, pl.ds(cc * 16, 16)] = jnp.zeros((16,), jnp.float32)
        return 0

    lax.fori_loop(0, WCH, zero_body, 0)
    for i in range(R_T // WCH):
        pltpu.sync_copy(gbuf_a.at[pl.ds(0, WCH)],
                        acc_sh.at[pl.ds(s * R_T + i * WCH, WCH)])
    plsc.subcore_barrier()

    def idx_rows(g):
        return pl.ds((sup0 + g) * SUB, SUB)

    def gather(ib, m, buf, sem):
        pltpu.async_copy(x_hbm.at[ib.at[m]], buf, sem)

    def gather_wait(ib, m, buf, sem):
        pltpu.make_async_copy(x_hbm.at[ib.at[m]], buf, sem).wait()

    def scatter_add(ib, m, buf):
        pltpu.sync_copy(buf, acc_sh.at[ib.at[m]], add=True)

    ring = ((gbuf_a, sem_a), (gbuf_b, sem_b), (gbuf_c, sem_c))

    def run(n_sup):
        pltpu.sync_copy(srcc_hbm.at[idx_rows(0)], is_a)
        pltpu.sync_copy(dstc_hbm.at[idx_rows(0)], id_a)
        for g in range(n_sup):
            isc, idc, isn, idn = ((is_a, id_a, is_b, id_b) if g % 2 == 0
                                  else (is_b, id_b, is_a, id_a))
            if g + 1 < n_sup:
                pltpu.async_copy(srcc_hbm.at[idx_rows(g + 1)], isn, sem_i)
                pltpu.async_copy(dstc_hbm.at[idx_rows(g + 1)], idn, sem_i)

            for t in range(3):
                gather(isc, t, *ring[t])

            def chunk_body(k, _, isc=isc, idc=idc):
                m0 = 3 * k
                for t in range(3):
                    gather_wait(isc, m0 + t, *ring[t])
                    scatter_add(idc, m0 + t, ring[t][0])
                    gather(isc, m0 + t + 3, *ring[t])
                return 0

            lax.fori_loop(0, (SUB - 3) // 3, chunk_body, 0)

            for t in range(3):
                gather_wait(isc, SUB - 3 + t, *ring[t])
                scatter_add(idc, SUB - 3 + t, ring[t][0])

            if g + 1 < n_sup:
                pltpu.make_async_copy(srcc_hbm.at[idx_rows(g + 1)], isn,
                                      sem_i).wait()
                pltpu.make_async_copy(dstc_hbm.at[idx_rows(g + 1)], idn,
                                      sem_i).wait()

    @pl.when(c == SLOW_CORE)
    def _():
        run(K_SLOW)

    @pl.when(c != SLOW_CORE)
    def _():
        run(K_FAST)

    plsc.subcore_barrier()

    for i in range(R_T // WCH):
        r0 = s * R_T + i * WCH
        pltpu.sync_copy(acc_sh.at[pl.ds(r0, WCH)], gbuf_a.at[pl.ds(0, WCH)])
        pltpu.sync_copy(gbuf_a.at[pl.ds(0, WCH)], out_hbm.at[c, pl.ds(r0, WCH)])


_agg = functools.partial(
    pl.kernel,
    out_type=jax.ShapeDtypeStruct((NC, R_ACC, D), jnp.float32),
    mesh=plsc.VectorSubcoreMesh(core_axis_name="c", subcore_axis_name="s",
                                num_cores=NC, num_subcores=NS),
    scratch_types=[
        pltpu.VMEM((SUB, CH), jnp.int32),
        pltpu.VMEM((SUB, CH), jnp.int32),
        pltpu.VMEM((SUB, CH), jnp.int32),
        pltpu.VMEM((SUB, CH), jnp.int32),
        pltpu.VMEM((CH, D), jnp.float32),
        pltpu.VMEM((CH, D), jnp.float32),
        pltpu.VMEM((CH, D), jnp.float32),
        pltpu.VMEM_SHARED((R_ACC, D), jnp.float32),
        pltpu.SemaphoreType.DMA,
        pltpu.SemaphoreType.DMA,
        pltpu.SemaphoreType.DMA,
        pltpu.SemaphoreType.DMA,
    ],
)(_agg_body)


def _mlp_body(x_ref, a0_ref, a1_ref, w1t_ref, b1_ref, w2t_ref, b2_ref, o_ref):
    h = x_ref[...] + a0_ref[0] + a1_ref[0]
    h = jnp.dot(h, w1t_ref[...], preferred_element_type=jnp.float32)
    h = jnp.maximum(h + b1_ref[...], 0.0)
    o_ref[...] = (jnp.dot(h, w2t_ref[...], preferred_element_type=jnp.float32)
                  + b2_ref[...])


def _mlp(x, agg, w1t, b1, w2t, b2):
    blk = 2000
    grid = (N_NODES // blk,)
    row_spec = pl.BlockSpec((blk, D), lambda i: (i, 0))
    agg0_spec = pl.BlockSpec((1, blk, D), lambda i: (0, i, 0))
    agg1_spec = pl.BlockSpec((1, blk, D), lambda i: (1, i, 0))
    full = pl.BlockSpec((D, D), lambda i: (0, 0))
    bias = pl.BlockSpec((1, D), lambda i: (0, 0))
    return pl.pallas_call(
        _mlp_body,
        grid=grid,
        in_specs=[row_spec, agg0_spec, agg1_spec, full, bias, full, bias],
        out_specs=row_spec,
        out_shape=jax.ShapeDtypeStruct((N_NODES, D), jnp.float32),
        compiler_params=pltpu.CompilerParams(
            dimension_semantics=("arbitrary",)),
    )(x, agg, agg, w1t, b1, w2t, b2)


def kernel(x, edge_index, W1, b1, W2, b2):
    src = edge_index[0].astype(jnp.int32)
    dst = edge_index[1].astype(jnp.int32)
    n_edges = src.shape[0]
    pad = E_PAD - n_edges
    dummy = DUMMY + jnp.arange(pad, dtype=jnp.int32) % (R_ACC - N_NODES)
    src = jnp.concatenate([src, jnp.zeros((pad,), jnp.int32)])
    dst = jnp.concatenate([dst, dummy])
    srcc = src.reshape(N_CH_TOT, CH)
    dstc = dst.reshape(N_CH_TOT, CH)

    agg = _agg(x, srcc, dstc)

    return _mlp(x, agg, W1.T, b1.reshape(1, D), W2.T, b2.reshape(1, D))


# speedup vs baseline: 1.5231x; 1.5231x over previous
import functools

import jax
import jax.numpy as jnp
from jax import lax
from jax.experimental import pallas as pl
from jax.experimental.pallas import tpu as pltpu
from jax.experimental.pallas import tpu_sc as plsc

N_NODES = 10000
D = 128

NC = 2
NS = 16
NW = NC * NS

CH = 128
SUB = 8
K_SUP = 10
N_CH_TOT = NW * K_SUP * SUB
E_PAD = N_CH_TOT * CH
R_ACC = 10112
R_T = R_ACC // NS
WCHS = (128, 128, 128, 128, 120)
DUMMY = N_NODES


def _agg_body(x_hbm, srcc_hbm, dstc_hbm, out_hbm,
              is_a, is_b, id_a, id_b, gbuf_a, gbuf_b, acc_sh,
              sem_a, sem_b, sem_i):
    c = lax.axis_index("c")
    s = lax.axis_index("s")
    wid = s * NC + c
    sup0 = wid * K_SUP

    def zero_body(i, _):
        for cc in range(D // 16):
            gbuf_a[i, pl.ds(cc * 16, 16)] = jnp.zeros((16,), jnp.float32)
        return 0

    lax.fori_loop(0, CH, zero_body, 0)
    off = 0
    for w in WCHS:
        pltpu.sync_copy(gbuf_a.at[pl.ds(0, w)],
                        acc_sh.at[pl.ds(s * R_T + off, w)])
        off += w
    plsc.subcore_barrier()

    def idx_rows(g):
        return pl.ds((sup0 + g) * SUB, SUB)

    def gather(ib, m, buf, sem):
        pltpu.async_copy(x_hbm.at[ib.at[m]], buf, sem)

    def gather_wait(ib, m, buf, sem):
        pltpu.make_async_copy(x_hbm.at[ib.at[m]], buf, sem).wait()

    def scatter_add(ib, m, buf):
        pltpu.sync_copy(buf, acc_sh.at[ib.at[m]], add=True)

    pltpu.sync_copy(srcc_hbm.at[idx_rows(0)], is_a)
    pltpu.sync_copy(dstc_hbm.at[idx_rows(0)], id_a)
    gather(is_a, 0, gbuf_a, sem_a)
    for g in range(K_SUP):
        isc, idc, isn, idn = ((is_a, id_a, is_b, id_b) if g % 2 == 0
                              else (is_b, id_b, is_a, id_a))
        if g + 1 < K_SUP:
            pltpu.async_copy(srcc_hbm.at[idx_rows(g + 1)], isn, sem_i)
            pltpu.async_copy(dstc_hbm.at[idx_rows(g + 1)], idn, sem_i)

        def chunk_body(k, _, isc=isc, idc=idc):
            m0 = 2 * k
            gather(isc, m0 + 1, gbuf_b, sem_b)
            gather_wait(isc, m0, gbuf_a, sem_a)
            scatter_add(idc, m0, gbuf_a)
            gather(isc, m0 + 2, gbuf_a, sem_a)
            gather_wait(isc, m0 + 1, gbuf_b, sem_b)
            scatter_add(idc, m0 + 1, gbuf_b)
            return 0

        lax.fori_loop(0, SUB // 2 - 1, chunk_body, 0)

        gather(isc, SUB - 1, gbuf_b, sem_b)
        gather_wait(isc, SUB - 2, gbuf_a, sem_a)
        scatter_add(idc, SUB - 2, gbuf_a)
        if g + 1 < K_SUP:
            pltpu.make_async_copy(srcc_hbm.at[idx_rows(g + 1)], isn,
                                  sem_i).wait()
            pltpu.make_async_copy(dstc_hbm.at[idx_rows(g + 1)], idn,
                                  sem_i).wait()
            gather(isn, 0, gbuf_a, sem_a)
        gather_wait(isc, SUB - 1, gbuf_b, sem_b)
        scatter_add(idc, SUB - 1, gbuf_b)

    plsc.subcore_barrier()

    off = 0
    for w in WCHS:
        r0 = s * R_T + off
        pltpu.sync_copy(acc_sh.at[pl.ds(r0, w)], gbuf_a.at[pl.ds(0, w)])
        pltpu.sync_copy(gbuf_a.at[pl.ds(0, w)], out_hbm.at[c, pl.ds(r0, w)])
        off += w


_agg = functools.partial(
    pl.kernel,
    out_type=jax.ShapeDtypeStruct((NC, R_ACC, D), jnp.float32),
    mesh=plsc.VectorSubcoreMesh(core_axis_name="c", subcore_axis_name="s",
                                num_cores=NC, num_subcores=NS),
    scratch_types=[
        pltpu.VMEM((SUB, CH), jnp.int32),
        pltpu.VMEM((SUB, CH), jnp.int32),
        pltpu.VMEM((SUB, CH), jnp.int32),
        pltpu.VMEM((SUB, CH), jnp.int32),
        pltpu.VMEM((CH, D), jnp.float32),
        pltpu.VMEM((CH, D), jnp.float32),
        pltpu.VMEM_SHARED((R_ACC, D), jnp.float32),
        pltpu.SemaphoreType.DMA,
        pltpu.SemaphoreType.DMA,
        pltpu.SemaphoreType.DMA,
    ],
)(_agg_body)


def _mlp_body(x_ref, a0_ref, a1_ref, w1t_ref, b1_ref, w2t_ref, b2_ref, o_ref):
    h = x_ref[...] + a0_ref[0] + a1_ref[0]
    h = jnp.dot(h, w1t_ref[...], preferred_element_type=jnp.float32)
    h = jnp.maximum(h + b1_ref[...], 0.0)
    o_ref[...] = (jnp.dot(h, w2t_ref[...], preferred_element_type=jnp.float32)
                  + b2_ref[...])


def _mlp(x, agg, w1t, b1, w2t, b2):
    blk = 2000
    grid = (N_NODES // blk,)
    row_spec = pl.BlockSpec((blk, D), lambda i: (i, 0))
    agg0_spec = pl.BlockSpec((1, blk, D), lambda i: (0, i, 0))
    agg1_spec = pl.BlockSpec((1, blk, D), lambda i: (1, i, 0))
    full = pl.BlockSpec((D, D), lambda i: (0, 0))
    bias = pl.BlockSpec((1, D), lambda i: (0, 0))
    return pl.pallas_call(
        _mlp_body,
        grid=grid,
        in_specs=[row_spec, agg0_spec, agg1_spec, full, bias, full, bias],
        out_specs=row_spec,
        out_shape=jax.ShapeDtypeStruct((N_NODES, D), jnp.float32),
        compiler_params=pltpu.CompilerParams(
            dimension_semantics=("arbitrary",)),
    )(x, agg, agg, w1t, b1, w2t, b2)


def kernel(x, edge_index, W1, b1, W2, b2):
    src = edge_index[0].astype(jnp.int32)
    dst = edge_index[1].astype(jnp.int32)
    n_edges = src.shape[0]
    pad = E_PAD - n_edges
    dummy = DUMMY + jnp.arange(pad, dtype=jnp.int32) % (R_ACC - N_NODES)
    src = jnp.concatenate([src, jnp.zeros((pad,), jnp.int32)])
    dst = jnp.concatenate([dst, dummy])
    srcc = src.reshape(N_CH_TOT, CH)
    dstc = dst.reshape(N_CH_TOT, CH)

    agg = _agg(x, srcc, dstc)

    return _mlp(x, agg, W1.T, b1.reshape(1, D), W2.T, b2.reshape(1, D))


# speedup vs baseline: 2.5810x; 1.6945x over previous
import functools

import jax
import jax.numpy as jnp
from jax import lax
from jax.experimental import pallas as pl
from jax.experimental.pallas import tpu as pltpu
from jax.experimental.pallas import tpu_sc as plsc

N_NODES = 10000
D = 128

NC = 2
NS = 16
NW = NC * NS

CH = 128
SUB = 8
K_SUP = 10
N_CH_TOT = NW * K_SUP * SUB
E_PAD = N_CH_TOT * CH
R_ACC = 10112
R_T = R_ACC // NS
WCHS = (128, 128, 128, 128, 120)
DUMMY = N_NODES


def _agg_body(x_hbm, srcc_hbm, dstc_hbm, out_hbm,
              is_a, is_b, id_a, id_b, gbuf_a, gbuf_b, fbuf, acc_sh,
              sem_a, sem_b, sem_i):
    c = lax.axis_index("c")
    s = lax.axis_index("s")
    wid = s * NC + c
    sup0 = wid * K_SUP

    def zero_body(i, _):
        for cc in range(D // 16):
            fbuf[i, pl.ds(cc * 16, 16)] = jnp.zeros((16,), jnp.float32)
        return 0

    lax.fori_loop(0, CH, zero_body, 0)
    off = 0
    for w in WCHS:
        pltpu.sync_copy(fbuf.at[pl.ds(0, w)],
                        acc_sh.at[pl.ds(s * R_T + off, w)])
        off += w
    plsc.subcore_barrier()

    def idx_rows(g):
        return pl.ds((sup0 + g) * SUB, SUB)

    def gather(ib, m, buf, sem):
        pltpu.async_copy(x_hbm.at[ib.at[m]], buf, sem)

    def gather_wait(ib, m, buf, sem):
        pltpu.make_async_copy(x_hbm.at[ib.at[m]], buf, sem).wait()

    def conv_scatter_add(ib, m, buf):
        def conv_body(i, _):
            for b in range(D // 32):
                w = buf[i, pl.ds(16 * b, 16)]
                lo = lax.bitcast_convert_type(w << 16, jnp.float32)
                hi = lax.bitcast_convert_type(w & jnp.int32(-65536),
                                              jnp.float32)
                fbuf[i, pl.ds(32 * b, 16)] = lo
                fbuf[i, pl.ds(32 * b + 16, 16)] = hi
            return 0

        lax.fori_loop(0, CH, conv_body, 0)
        pltpu.sync_copy(fbuf, acc_sh.at[ib.at[m]], add=True)

    pltpu.sync_copy(srcc_hbm.at[idx_rows(0)], is_a)
    pltpu.sync_copy(dstc_hbm.at[idx_rows(0)], id_a)
    gather(is_a, 0, gbuf_a, sem_a)
    for g in range(K_SUP):
        isc, idc, isn, idn = ((is_a, id_a, is_b, id_b) if g % 2 == 0
                              else (is_b, id_b, is_a, id_a))
        if g + 1 < K_SUP:
            pltpu.async_copy(srcc_hbm.at[idx_rows(g + 1)], isn, sem_i)
            pltpu.async_copy(dstc_hbm.at[idx_rows(g + 1)], idn, sem_i)

        def chunk_body(k, _, isc=isc, idc=idc):
            m0 = 2 * k
            gather(isc, m0 + 1, gbuf_b, sem_b)
            gather_wait(isc, m0, gbuf_a, sem_a)
            conv_scatter_add(idc, m0, gbuf_a)
            gather(isc, m0 + 2, gbuf_a, sem_a)
            gather_wait(isc, m0 + 1, gbuf_b, sem_b)
            conv_scatter_add(idc, m0 + 1, gbuf_b)
            return 0

        lax.fori_loop(0, SUB // 2 - 1, chunk_body, 0)

        gather(isc, SUB - 1, gbuf_b, sem_b)
        gather_wait(isc, SUB - 2, gbuf_a, sem_a)
        conv_scatter_add(idc, SUB - 2, gbuf_a)
        if g + 1 < K_SUP:
            pltpu.make_async_copy(srcc_hbm.at[idx_rows(g + 1)], isn,
                                  sem_i).wait()
            pltpu.make_async_copy(dstc_hbm.at[idx_rows(g + 1)], idn,
                                  sem_i).wait()
            gather(isn, 0, gbuf_a, sem_a)
        gather_wait(isc, SUB - 1, gbuf_b, sem_b)
        conv_scatter_add(idc, SUB - 1, gbuf_b)

    plsc.subcore_barrier()

    off = 0
    for w in WCHS:
        r0 = s * R_T + off
        pltpu.sync_copy(acc_sh.at[pl.ds(r0, w)], fbuf.at[pl.ds(0, w)])
        pltpu.sync_copy(fbuf.at[pl.ds(0, w)], out_hbm.at[c, pl.ds(r0, w)])
        off += w


_agg = functools.partial(
    pl.kernel,
    out_type=jax.ShapeDtypeStruct((NC, R_ACC, D), jnp.float32),
    mesh=plsc.VectorSubcoreMesh(core_axis_name="c", subcore_axis_name="s",
                                num_cores=NC, num_subcores=NS),
    scratch_types=[
        pltpu.VMEM((SUB, CH), jnp.int32),
        pltpu.VMEM((SUB, CH), jnp.int32),
        pltpu.VMEM((SUB, CH), jnp.int32),
        pltpu.VMEM((SUB, CH), jnp.int32),
        pltpu.VMEM((CH, D // 2), jnp.int32),
        pltpu.VMEM((CH, D // 2), jnp.int32),
        pltpu.VMEM((CH, D), jnp.float32),
        pltpu.VMEM_SHARED((R_ACC, D), jnp.float32),
        pltpu.SemaphoreType.DMA,
        pltpu.SemaphoreType.DMA,
        pltpu.SemaphoreType.DMA,
    ],
    compiler_params=pltpu.CompilerParams(use_tc_tiling_on_sc=False),
)(_agg_body)


def _mlp_body(x_ref, a0_ref, a1_ref, w1t_ref, b1_ref, w2t_ref, b2_ref, o_ref):
    h = x_ref[...] + a0_ref[0] + a1_ref[0]
    h = jnp.dot(h, w1t_ref[...], preferred_element_type=jnp.float32)
    h = jnp.maximum(h + b1_ref[...], 0.0)
    o_ref[...] = (jnp.dot(h, w2t_ref[...], preferred_element_type=jnp.float32)
                  + b2_ref[...])


def _mlp(x, agg, w1t, b1, w2t, b2):
    blk = 2000
    grid = (N_NODES // blk,)
    row_spec = pl.BlockSpec((blk, D), lambda i: (i, 0))
    agg0_spec = pl.BlockSpec((1, blk, D), lambda i: (0, i, 0))
    agg1_spec = pl.BlockSpec((1, blk, D), lambda i: (1, i, 0))
    full = pl.BlockSpec((D, D), lambda i: (0, 0))
    bias = pl.BlockSpec((1, D), lambda i: (0, 0))
    return pl.pallas_call(
        _mlp_body,
        grid=grid,
        in_specs=[row_spec, agg0_spec, agg1_spec, full, bias, full, bias],
        out_specs=row_spec,
        out_shape=jax.ShapeDtypeStruct((N_NODES, D), jnp.float32),
        compiler_params=pltpu.CompilerParams(
            dimension_semantics=("arbitrary",)),
    )(x, agg, agg, w1t, b1, w2t, b2)


def kernel(x, edge_index, W1, b1, W2, b2):
    src = edge_index[0].astype(jnp.int32)
    dst = edge_index[1].astype(jnp.int32)
    n_edges = src.shape[0]
    pad = E_PAD - n_edges
    dummy = DUMMY + jnp.arange(pad, dtype=jnp.int32) % (R_ACC - N_NODES)
    src = jnp.concatenate([src, jnp.zeros((pad,), jnp.int32)])
    dst = jnp.concatenate([dst, dummy])
    srcc = src.reshape(N_CH_TOT, CH)
    dstc = dst.reshape(N_CH_TOT, CH)

    x16 = (x.astype(jnp.bfloat16)
            .reshape(N_NODES, D // 32, 2, 16)
            .transpose(0, 1, 3, 2)
            .reshape(N_NODES, D // 2, 2))
    x16i = jax.lax.bitcast_convert_type(x16, jnp.int32)

    agg = _agg(x16i, srcc, dstc)

    return _mlp(x, agg, W1.T, b1.reshape(1, D), W2.T, b2.reshape(1, D))
